# Initial kernel scaffold; baseline (speedup 1.0000x reference)
import functools

import jax
import jax.numpy as jnp
from jax import lax
from jax.experimental import pallas as pl
from jax.experimental.pallas import tpu as pltpu
from jax.experimental.pallas import tpu_sc as plsc

H = 4
D = 16
GO = H * D
NC = 2
NS = 16
NW = NC * NS
C = 128
QUANT = jnp.float32(0.1)


def _pre_body(h_ref, t_ref, wrW_ref, wrb_ref, weW_ref, web_ref,
              Wfc_ref, Wl_ref, Wr_ref, Wres_ref,
              ft_ref, el_ref, er_ref, res_ref):
    t = t_ref[...]
    tq = jnp.floor(t / QUANT) * QUANT
    red = jnp.cos(tq * wrW_ref[...] + wrb_ref[...])
    red = jnp.maximum(red, 0.0)
    temb = jnp.dot(red, weW_ref[...], preferred_element_type=jnp.float32)
    temb = temb + web_ref[...]
    feat = jnp.concatenate([h_ref[...], temb], axis=1)
    ft = jnp.dot(feat, Wfc_ref[...], preferred_element_type=jnp.float32)
    ft_ref[...] = ft
    el_ref[...] = jnp.dot(ft, Wl_ref[...], preferred_element_type=jnp.float32)
    er_ref[...] = jnp.dot(ft, Wr_ref[...], preferred_element_type=jnp.float32)
    res_ref[...] = jnp.dot(feat, Wres_ref[...],
                           preferred_element_type=jnp.float32)


def _post_body(parts_ref, res_ref, Wp_ref, bp_ref, out_ref):
    r = parts_ref[0] + parts_ref[1] + res_ref[...]
    r = jnp.where(r > 0, r, jnp.expm1(r))
    out_ref[...] = jnp.dot(r, Wp_ref[...],
                           preferred_element_type=jnp.float32) + bp_ref[...]


def _sc_edge(NP, K, ft_hbm, elf_hbm, erf_hbm, src_hbm, dst_hbm, zero_hbm,
             out_hbm, el_tab, er_tab, ftrows, idxs, idxd, abuf, acc, gsem):
    cid = lax.axis_index("c")
    sid = lax.axis_index("s")
    wid = cid * NS + sid
    rpt = NP // NS
    pltpu.sync_copy(zero_hbm.at[pl.ds(sid * rpt, rpt)],
                    acc.at[pl.ds(sid * rpt, rpt)])
    pltpu.sync_copy(elf_hbm, el_tab)
    pltpu.sync_copy(erf_hbm, er_tab)
    plsc.subcore_barrier()

    PW = K * C

    def chunk(k, carry):
        base = wid * PW + k * C
        pltpu.sync_copy(src_hbm.at[pl.ds(base, C)], idxs)
        pltpu.sync_copy(dst_hbm.at[pl.ds(base, C)], idxd)
        pltpu.async_copy(ft_hbm.at[idxs], ftrows, gsem).wait()
        for g in range(C // 16):
            sv = idxs[pl.ds(g * 16, 16)]
            dv = idxd[pl.ds(g * 16, 16)]
            logits = []
            for hh in range(H):
                elv = plsc.load_gather(el_tab, [sv + hh * NP])
                erv = plsc.load_gather(er_tab, [dv + hh * NP])
                s = elv + erv
                logits.append(jnp.where(s > 0, s, 0.01 * s))
            m = jnp.maximum(jnp.maximum(logits[0], logits[1]),
                            jnp.maximum(logits[2], logits[3]))
            exps = [jnp.exp(l - m) for l in logits]
            inv = 1.0 / (exps[0] + exps[1] + exps[2] + exps[3])
            for hh in range(H):
                abuf[hh, :] = exps[hh] * inv
            for i in range(16):
                r = g * 16 + i
                for hh in range(H):
                    w = abuf[hh, i]
                    ftrows[r, pl.ds(hh * D, D)] = \
                        ftrows[r, pl.ds(hh * D, D)] * w
        pltpu.sync_copy(ftrows, acc.at[idxd], add=True)
        return carry

    lax.fori_loop(0, K, chunk, None)
    plsc.subcore_barrier()
    pltpu.sync_copy(acc.at[pl.ds(sid * rpt, rpt)],
                    out_hbm.at[cid, pl.ds(sid * rpt, rpt)])


def kernel(h, edge_index, t, W_fc, attn_l, attn_r, W_res,
           w_reduce_W, w_reduce_b, w_expand_W, w_expand_b, W_proj, b_proj):
    N, IN_DIM = h.shape
    E = edge_index.shape[1]
    RED = w_reduce_W.shape[1]
    TD = w_expand_W.shape[1]
    OUT = W_proj.shape[1]

    NP = ((N + 128) // 128) * 128
    K = -(-E // (NW * C))
    EP = NW * K * C

    f32 = jnp.float32
    src = jnp.pad(edge_index[0].astype(jnp.int32), (0, EP - E),
                  constant_values=N)
    dst = jnp.pad(edge_index[1].astype(jnp.int32), (0, EP - E),
                  constant_values=N)
    h_p = jnp.pad(h, ((0, NP - N), (0, 0)))
    t_p = jnp.pad(t, (0, NP - N)).reshape(NP, 1)

    A_l = attn_l.reshape(H, D)
    A_r = attn_r.reshape(H, D)
    eye = jnp.eye(H, dtype=f32)
    Wl = (eye[:, None, :] * A_l[:, :, None]).reshape(H * D, H)
    Wr = (eye[:, None, :] * A_r[:, :, None]).reshape(H * D, H)

    pre = pl.pallas_call(
        _pre_body,
        out_shape=[
            jax.ShapeDtypeStruct((NP, GO), f32),
            jax.ShapeDtypeStruct((NP, H), f32),
            jax.ShapeDtypeStruct((NP, H), f32),
            jax.ShapeDtypeStruct((NP, GO), f32),
        ],
    )
    ft, el, er, res = pre(h_p, t_p,
                          w_reduce_W, w_reduce_b.reshape(1, RED),
                          w_expand_W, w_expand_b.reshape(1, TD),
                          W_fc, Wl, Wr, W_res)

    elf = el.T.reshape(-1)
    erf = er.T.reshape(-1)
    zero = jnp.zeros((NP, GO), f32)

    mesh = plsc.VectorSubcoreMesh(core_axis_name="c", subcore_axis_name="s")
    edge_fn = pl.kernel(
        functools.partial(_sc_edge, NP, K),
        out_type=jax.ShapeDtypeStruct((NC, NP, GO), f32),
        mesh=mesh,
        scratch_types=[
            pltpu.VMEM((H * NP,), f32),
            pltpu.VMEM((H * NP,), f32),
            pltpu.VMEM((C, GO), f32),
            pltpu.VMEM((C,), jnp.int32),
            pltpu.VMEM((C,), jnp.int32),
            pltpu.VMEM((H, 16), f32),
            pltpu.VMEM_SHARED((NP, GO), f32),
            pltpu.SemaphoreType.DMA,
        ],
    )
    parts = edge_fn(ft, elf, erf, src, dst, zero)

    post = pl.pallas_call(
        _post_body,
        out_shape=jax.ShapeDtypeStruct((NP, OUT), f32),
    )
    out = post(parts, res, W_proj, b_proj.reshape(1, OUT))
    return out[:N]


# speedup vs baseline: 29.2955x; 29.2955x over previous
import functools

import jax
import jax.numpy as jnp
from jax import lax
from jax.experimental import pallas as pl
from jax.experimental.pallas import tpu as pltpu
from jax.experimental.pallas import tpu_sc as plsc

H = 4
HD = 16
GO = H * HD
W = 128
NC = 2
NS = 16
NW = NC * NS
C = 64
PK = 16384
QUANT = 0.1


def _pre_body(h_ref, t_ref, wrW_ref, wrb_ref, weW_ref, web_ref,
              Wfc_ref, Wl_ref, Wr_ref, Wres_ref,
              ftp_ref, er_ref, res_ref):
    t = t_ref[...]
    tq = jnp.floor(t / QUANT) * QUANT
    red = jnp.cos(tq * wrW_ref[...] + wrb_ref[...])
    red = jnp.maximum(red, 0.0)
    temb = jnp.dot(red, weW_ref[...], preferred_element_type=jnp.float32)
    temb = temb + web_ref[...]
    feat = jnp.concatenate([h_ref[...], temb], axis=1)
    ft = jnp.dot(feat, Wfc_ref[...], preferred_element_type=jnp.float32)
    el = jnp.dot(ft, Wl_ref[...], preferred_element_type=jnp.float32)
    np_ = ft.shape[0]
    ftp_ref[...] = jnp.concatenate(
        [ft, el, jnp.zeros((np_, W - GO - H), jnp.float32)], axis=1)
    er_ref[...] = jnp.dot(ft, Wr_ref[...], preferred_element_type=jnp.float32)
    res_ref[...] = jnp.dot(feat, Wres_ref[...],
                           preferred_element_type=jnp.float32)


def _post_body(parts_ref, res_ref, Wp_ref, bp_ref, out_ref):
    np_ = res_ref.shape[0]
    agg = (parts_ref[0, :np_, :] + parts_ref[1, :np_, :]) + res_ref[...]
    r = jnp.where(agg > 0, agg, jnp.exp(jnp.minimum(agg, 0.0)) - 1.0)
    out_ref[...] = jnp.dot(r, Wp_ref[...],
                           preferred_element_type=jnp.float32) + bp_ref[...]


def _sc_edge(NP, K, ftp_hbm, erf_hbm, edge_hbm, out_hbm,
             er0, er1, er2, er3, stamp, ftrows, wrows, idxe, idxs, sidx,
             acc, gsem):
    cid = lax.axis_index("c")
    sid = lax.axis_index("s")
    wid = cid * NS + sid
    NPH = NP // 2
    AR = NPH + 128
    rpt = AR // NS
    SR = stamp.shape[0]
    er_tabs = [er0, er1, er2, er3]
    zero16 = jnp.zeros((16,), jnp.float32)
    izero16 = jnp.zeros((16,), jnp.int32)
    iota16 = lax.iota(jnp.int32, 16)
    for s in range(C):
        for q in range(W // 16):
            wrows[s, pl.ds(q * 16, 16)] = zero16
    def zrow(j, carry):
        pltpu.sync_copy(wrows, acc.at[pl.ds(sid * rpt + j * C, C)])
        return carry
    lax.fori_loop(0, rpt // C, zrow, None)
    if rpt % C:
        pltpu.sync_copy(wrows.at[pl.ds(0, rpt % C)],
                        acc.at[pl.ds(sid * rpt + (rpt // C) * C, rpt % C)])
    def zst(j, carry):
        stamp[pl.ds(j * 16, 16)] = izero16
        return carry
    lax.fori_loop(0, SR // 16, zst, None)
    for hh in range(H):
        pltpu.sync_copy(erf_hbm.at[pl.ds(hh * NP, NP)], er_tabs[hh])
    plsc.subcore_barrier()

    PW = K * C

    def chunk(k, carry):
        base = wid * PW + k * C
        pltpu.sync_copy(edge_hbm.at[pl.ds(base, C)], idxe)
        for g in range(C // 16):
            ev = idxe[pl.ds(g * 16, 16)]
            idxs[pl.ds(g * 16, 16)] = lax.shift_right_logical(ev, 14)
        pltpu.async_copy(ftp_hbm.at[idxs], ftrows, gsem).wait()
        base_code = (k + 1) * C
        for s in range(C):
            for q in range(W // 16):
                wrows[s, pl.ds(q * 16, 16)] = zero16
        for g in range(C // 16):
            slotv = 16 * g + iota16
            ev = idxe[pl.ds(g * 16, 16)]
            dv = ev & (PK - 1)
            rv = lax.shift_right_logical(dv, 1)
            pv = (dv & 1) * GO
            svec = base_code + slotv
            cur = plsc.load_gather(stamp, [rv])
            fresh = cur < base_code
            plsc.store_scatter(stamp, [rv], svec, mask=fresh)
            got = plsc.load_gather(stamp, [rv])
            owner = got == svec
            tgt = got - base_code
            sidx[pl.ds(g * 16, 16)] = jnp.where(owner, rv, NPH + slotv)
            logits = []
            for hh in range(H):
                elv = plsc.load_gather(
                    ftrows, [slotv, jnp.full((16,), GO + hh, jnp.int32)])
                erv = plsc.load_gather(er_tabs[hh], [dv])
                s_ = elv + erv
                logits.append(jnp.where(s_ > 0, s_, 0.01 * s_))
            m = jnp.maximum(jnp.maximum(logits[0], logits[1]),
                            jnp.maximum(logits[2], logits[3]))
            exps = [jnp.exp(l - m) for l in logits]
            inv = 1.0 / (exps[0] + exps[1] + exps[2] + exps[3])
            aw = [e * inv for e in exps]
            for i in range(16):
                s_slot = g * 16 + i
                tvec = jnp.full((16,), tgt[i], jnp.int32)
                cb = pv[i] + iota16
                for hh in range(H):
                    wv = aw[hh][i]
                    row = ftrows[s_slot, pl.ds(hh * HD, HD)]
                    plsc.addupdate_scatter(
                        wrows, [tvec, cb + hh * HD], row * wv)
        pltpu.sync_copy(wrows, acc.at[sidx], add=True)
        return carry

    lax.fori_loop(0, K, chunk, None)
    plsc.subcore_barrier()
    pltpu.sync_copy(acc.at[pl.ds(sid * rpt, rpt)],
                    out_hbm.at[cid, pl.ds(sid * rpt, rpt)])


def kernel(h, edge_index, t, W_fc, attn_l, attn_r, W_res,
           w_reduce_W, w_reduce_b, w_expand_W, w_expand_b, W_proj, b_proj):
    N, IN_DIM = h.shape
    E = edge_index.shape[1]
    RED = w_reduce_W.shape[1]
    TD = w_expand_W.shape[1]
    OUT = W_proj.shape[1]

    NP = ((N + 256) // 256) * 256
    K = -(-E // (NW * C))
    EP = NW * K * C
    NPH = NP // 2
    AR = NPH + 128

    f32 = jnp.float32
    src = edge_index[0].astype(jnp.int32)
    dst = edge_index[1].astype(jnp.int32)
    epk = jnp.pad(src * PK + dst, (0, EP - E), constant_values=N * PK + N)
    h_p = jnp.pad(h, ((0, NP - N), (0, 0)))
    t_p = jnp.pad(t, (0, NP - N)).reshape(NP, 1)

    A_l = attn_l.reshape(H, HD)
    A_r = attn_r.reshape(H, HD)
    eye = jnp.eye(H, dtype=f32)
    Wl = (eye[:, None, :] * A_l[:, :, None]).reshape(H * HD, H)
    Wr = (eye[:, None, :] * A_r[:, :, None]).reshape(H * HD, H)

    pre = pl.pallas_call(
        _pre_body,
        out_shape=[
            jax.ShapeDtypeStruct((NP, W), f32),
            jax.ShapeDtypeStruct((NP, H), f32),
            jax.ShapeDtypeStruct((NP, GO), f32),
        ],
    )
    ftp, er, res = pre(h_p, t_p,
                       w_reduce_W, w_reduce_b.reshape(1, RED),
                       w_expand_W, w_expand_b.reshape(1, TD),
                       W_fc, Wl, Wr, W_res)

    erf = er.T.reshape(-1)

    mesh = plsc.VectorSubcoreMesh(core_axis_name="c", subcore_axis_name="s",
                                  num_cores=NC, num_subcores=NS)
    edge_fn = pl.kernel(
        functools.partial(_sc_edge, NP, K),
        out_type=jax.ShapeDtypeStruct((NC, AR, W), f32),
        mesh=mesh,
        compiler_params=pltpu.CompilerParams(needs_layout_passes=False),
        scratch_types=[
            pltpu.VMEM((NP,), f32),
            pltpu.VMEM((NP,), f32),
            pltpu.VMEM((NP,), f32),
            pltpu.VMEM((NP,), f32),
            pltpu.VMEM((NPH + 16,), jnp.int32),
            pltpu.VMEM((C, W), f32),
            pltpu.VMEM((C, W), f32),
            pltpu.VMEM((C,), jnp.int32),
            pltpu.VMEM((C,), jnp.int32),
            pltpu.VMEM((C,), jnp.int32),
            pltpu.VMEM_SHARED((AR, W), f32),
            pltpu.SemaphoreType.DMA,
        ],
    )
    parts = edge_fn(ftp, erf, epk)

    parts2 = parts.reshape(NC, 2 * AR, GO)

    post = pl.pallas_call(
        _post_body,
        out_shape=jax.ShapeDtypeStruct((NP, OUT), f32),
    )
    out = post(parts2, res, W_proj, b_proj.reshape(1, OUT))
    return out[:N]
